# baseline (speedup 1.0000x reference)
import functools

import jax
import jax.numpy as jnp
from jax import lax
from jax.experimental import pallas as pl
from jax.experimental.pallas import tpu as pltpu
from jax.experimental.pallas import tpu_sc as plsc

VOCAB = 1000000
EMBED_DIM = 64
BATCH = 4096
HIST = 200
HIDDEN = 512
OUT = 128

NC = 2
NS = 16
NW = NC * NS
ROWS_PER_W = BATCH // NW

CHUNK = 128
N_FULL_CHUNKS = VOCAB // CHUNK
TAIL_ROWS = VOCAB - N_FULL_CHUNKS * CHUNK
BASE_CHUNKS = N_FULL_CHUNKS // NW
EXTRA_CHUNKS = N_FULL_CHUNKS % NW
PACKED_ROWS = VOCAB // 2


def _sc_depad(tableT, tail128):
    mesh = plsc.VectorSubcoreMesh(core_axis_name="c", subcore_axis_name="s")

    @functools.partial(
        pl.kernel,
        out_type=jax.ShapeDtypeStruct((PACKED_ROWS, 128), jnp.float32),
        mesh=mesh,
        scratch_types=[
            pltpu.VMEM((6, EMBED_DIM, CHUNK), jnp.float32),
            pltpu.VMEM((3, CHUNK // 2, 2 * EMBED_DIM), jnp.float32),
            pltpu.VMEM((TAIL_ROWS // 2, 128), jnp.float32),
            pltpu.SemaphoreType.DMA,
            pltpu.SemaphoreType.DMA,
            pltpu.SemaphoreType.DMA,
            pltpu.SemaphoreType.DMA,
            pltpu.SemaphoreType.DMA,
            pltpu.SemaphoreType.DMA,
            pltpu.SemaphoreType.DMA,
            pltpu.SemaphoreType.DMA,
            pltpu.SemaphoreType.DMA,
            pltpu.SemaphoreType.DMA,
        ],
        compiler_params=pltpu.CompilerParams(
            use_tc_tiling_on_sc=True, needs_layout_passes=False),
    )
    def depad(tableT_hbm, tail_hbm, out_hbm, ibuf, obuf, tbuf,
              rsem0, rsem1, rsem2, rsem3, rsem4, rsem5,
              wsem0, wsem1, wsem2, tsem):
        wid = lax.axis_index("s") * NC + lax.axis_index("c")
        nchunks = BASE_CHUNKS + jnp.where(wid < EXTRA_CHUNKS, 1, 0)
        rsems = (rsem0, rsem1, rsem2, rsem3, rsem4, rsem5)
        wsems = (wsem0, wsem1, wsem2)

        def chunk_id(j):
            return wid + NW * j

        def issue_read(j, slot):
            for b in range(EMBED_DIM // 8):
                pltpu.async_copy(
                    tableT_hbm.at[pl.ds(8 * b, 8),
                                  pl.ds(chunk_id(j) * CHUNK, CHUNK)],
                    ibuf.at[slot, pl.ds(8 * b, 8)], rsems[slot])

        def drain_read(j, slot):
            pltpu.make_async_copy(
                out_hbm.at[pl.ds(0, EMBED_DIM)], ibuf.at[slot],
                rsems[slot]).wait()

        def issue_write(j, slot):
            pltpu.async_copy(
                obuf.at[slot],
                out_hbm.at[pl.ds(chunk_id(j) * (CHUNK // 2), CHUNK // 2)],
                wsems[slot])

        def drain_write(j, slot):
            pltpu.make_async_copy(
                obuf.at[slot],
                out_hbm.at[pl.ds(chunk_id(j) * (CHUNK // 2), CHUNK // 2)],
                wsems[slot]).wait()

        @pl.when(wid == NW - 1)
        def _():
            pltpu.async_copy(tail_hbm, tbuf, tsem).wait()
            pltpu.async_copy(
                tbuf, out_hbm.at[pl.ds(N_FULL_CHUNKS * (CHUNK // 2),
                                       TAIL_ROWS // 2)], tsem).wait()

        row_iota = lax.iota(jnp.int32, 16)
        xor_idx = {k: jnp.bitwise_xor(row_iota, k) for k in (1, 2, 4, 8)}
        lane_masks = {k: (jnp.bitwise_and(row_iota, k) == 0)
                      for k in (1, 2, 4, 8)}

        shuf_dnums = lax.GatherDimensionNumbers(
            offset_dims=(), collapsed_slice_dims=(0,), start_index_map=(0,))

        def shuf(v, k):
            return lax.gather(
                v, xor_idx[k][:, None], shuf_dnums, slice_sizes=(1,),
                mode=lax.GatherScatterMode.PROMISE_IN_BOUNDS)

        for s in range(6):
            @pl.when(nchunks > s)
            def _(s=s):
                issue_read(s, s)

        def process(j, rslot, wslot):
            drain_read(j, rslot)

            @pl.when(j >= 3)
            def _():
                drain_write(j - 3, wslot)

            @plsc.parallel_loop(0, CHUNK // 16, unroll=1)
            def tloop(kb):
                for g in range(EMBED_DIM // 16):
                    a = [ibuf[rslot, 16 * g + j, pl.ds(16 * kb, 16)]
                         for j in range(16)]
                    for k in (1, 2, 4, 8):
                        m = lane_masks[k]
                        nxt = list(a)
                        for p in range(16):
                            if p & k:
                                continue
                            q = p | k
                            nxt[p] = jnp.where(m, a[p], shuf(a[q], k))
                            nxt[q] = jnp.where(m, shuf(a[p], k), a[q])
                        a = nxt
                    for i in range(16):
                        obuf[wslot, 8 * kb + i // 2,
                             pl.ds(EMBED_DIM * (i % 2) + 16 * g, 16)] = a[i]

            issue_write(j, wslot)

            @pl.when(j + 6 < nchunks)
            def _():
                issue_read(j + 6, rslot)

        @pl.loop(0, BASE_CHUNKS // 6 + 1)
        def body(t):
            for s in range(6):
                @pl.when(6 * t + s < nchunks)
                def _(s=s):
                    process(6 * t + s, s, s % 3)

        res = lax.rem(nchunks, 3)
        for r in range(3):
            @pl.when(res == r)
            def _(r=r):
                for t in range(3):
                    drain_write(nchunks - 3 + t, (r + t) % 3)

    return depad(tableT, tail128)


IDX_W = HIST
NBUF = 4


def _sc_pool(x2, table_lin):
    mesh = plsc.VectorSubcoreMesh(core_axis_name="c", subcore_axis_name="s")

    @functools.partial(
        pl.kernel,
        out_type=jax.ShapeDtypeStruct((BATCH, EMBED_DIM), jnp.float32),
        mesh=mesh,
        scratch_types=[
            pltpu.VMEM((ROWS_PER_W, IDX_W), jnp.int32),
            pltpu.VMEM((NBUF, IDX_W, EMBED_DIM), jnp.float32),
            pltpu.VMEM((ROWS_PER_W, EMBED_DIM), jnp.float32),
            pltpu.SemaphoreType.DMA,
            pltpu.SemaphoreType.DMA,
            pltpu.SemaphoreType.DMA,
            pltpu.SemaphoreType.DMA,
        ],
        compiler_params=pltpu.CompilerParams(use_tc_tiling_on_sc=False),
    )
    def pool(x_hbm, table_hbm, out_hbm, idx_v, gbuf, outbuf,
             sem0, sem1, sem2, sem3):
        wid = lax.axis_index("s") * NC + lax.axis_index("c")
        pltpu.sync_copy(
            x_hbm.at[pl.ds(wid * ROWS_PER_W, ROWS_PER_W)], idx_v)
        sems = (sem0, sem1, sem2, sem3)

        def issue(b, slot, sem):
            pltpu.async_copy(
                table_hbm.at[idx_v.at[b]], gbuf.at[slot], sem)

        for s in range(NBUF):
            issue(s, s, sems[s])

        @pl.loop(0, ROWS_PER_W, step=NBUF)
        def body(b):
            for s in range(NBUF):
                bb = b + s
                pltpu.make_async_copy(
                    table_hbm.at[idx_v.at[bb]], gbuf.at[s], sems[s]).wait()

                zero = jnp.zeros((16,), jnp.float32)

                def red(r, carry, s=s):
                    a0, a1, a2, a3 = carry
                    a0 = a0 + gbuf[s, r, 0:16]
                    a1 = a1 + gbuf[s, r, 16:32]
                    a2 = a2 + gbuf[s, r, 32:48]
                    a3 = a3 + gbuf[s, r, 48:64]
                    return (a0, a1, a2, a3)

                a0, a1, a2, a3 = pl.loop(
                    0, IDX_W, init_carry=(zero, zero, zero, zero),
                    unroll=8)(red)
                outbuf[bb, 0:16] = a0
                outbuf[bb, 16:32] = a1
                outbuf[bb, 32:48] = a2
                outbuf[bb, 48:64] = a3

                @pl.when(bb + NBUF < ROWS_PER_W)
                def _(bb=bb, s=s):
                    issue(bb + NBUF, s, sems[s])

        pltpu.sync_copy(outbuf, out_hbm.at[pl.ds(wid * ROWS_PER_W, ROWS_PER_W)])

    return pool(x2, table_lin)


BLK = 1024


def _mlp(x, sums, W1, b1, W2, b2):
    def mlp_body(x_ref, s_ref, w1_ref, b1_ref, w2_ref, b2_ref, o_ref):
        xi = x_ref[...]
        lengths = jnp.sum((xi != 0).astype(jnp.float32), axis=1, keepdims=True)
        pooled = s_ref[...] / lengths
        h = lax.dot_general(
            pooled, w1_ref[...], (((1,), (1,)), ((), ())),
            precision=lax.Precision.HIGHEST,
            preferred_element_type=jnp.float32) + b1_ref[...]
        h = jnp.maximum(h, 0.0)
        o_ref[...] = lax.dot_general(
            h, w2_ref[...], (((1,), (1,)), ((), ())),
            precision=lax.Precision.HIGHEST,
            preferred_element_type=jnp.float32) + b2_ref[...]

    return pl.pallas_call(
        mlp_body,
        grid=(BATCH // BLK,),
        in_specs=[
            pl.BlockSpec((BLK, HIST), lambda i: (i, 0)),
            pl.BlockSpec((BLK, EMBED_DIM), lambda i: (i, 0)),
            pl.BlockSpec((HIDDEN, EMBED_DIM), lambda i: (0, 0)),
            pl.BlockSpec((1, HIDDEN), lambda i: (0, 0)),
            pl.BlockSpec((OUT, HIDDEN), lambda i: (0, 0)),
            pl.BlockSpec((1, OUT), lambda i: (0, 0)),
        ],
        out_specs=pl.BlockSpec((BLK, OUT), lambda i: (i, 0)),
        out_shape=jax.ShapeDtypeStruct((BATCH, OUT), jnp.float32),
    )(x, sums, W1, b1.reshape(1, HIDDEN), W2, b2.reshape(1, OUT))


def kernel(x, table, W1, b1, W2, b2):
    tableT = table.T
    tail128 = table[N_FULL_CHUNKS * CHUNK:].reshape(TAIL_ROWS // 2, 128)
    packed = _sc_depad(tableT, tail128)
    table_lin = packed.reshape(VOCAB, EMBED_DIM)
    sums = _sc_pool(x, table_lin)
    return _mlp(x, sums, W1, b1, W2, b2)


# speedup vs baseline: 1.1969x; 1.1969x over previous
import functools

import jax
import jax.numpy as jnp
from jax import lax
from jax.experimental import pallas as pl
from jax.experimental.pallas import tpu as pltpu
from jax.experimental.pallas import tpu_sc as plsc

VOCAB = 1000000
EMBED_DIM = 64
BATCH = 4096
HIST = 200
HIDDEN = 512
OUT = 128

NC = 2
NS = 16
NW = NC * NS
ROWS_PER_W = BATCH // NW

CHUNK = 128
N_FULL_CHUNKS = VOCAB // CHUNK
TAIL_ROWS = VOCAB - N_FULL_CHUNKS * CHUNK
BASE_CHUNKS = N_FULL_CHUNKS // NW
EXTRA_CHUNKS = N_FULL_CHUNKS % NW
PACKED_ROWS = VOCAB // 2


def _sc_depad(tableT, tail128):
    mesh = plsc.VectorSubcoreMesh(core_axis_name="c", subcore_axis_name="s")

    @functools.partial(
        pl.kernel,
        out_type=jax.ShapeDtypeStruct((PACKED_ROWS, 128), jnp.float32),
        mesh=mesh,
        scratch_types=[
            pltpu.VMEM((3, EMBED_DIM, CHUNK), jnp.float32),
            pltpu.VMEM((3, CHUNK // 2, 2 * EMBED_DIM), jnp.float32),
            pltpu.VMEM((TAIL_ROWS // 2, 128), jnp.float32),
            pltpu.SemaphoreType.DMA,
            pltpu.SemaphoreType.DMA,
            pltpu.SemaphoreType.DMA,
            pltpu.SemaphoreType.DMA,
            pltpu.SemaphoreType.DMA,
            pltpu.SemaphoreType.DMA,
            pltpu.SemaphoreType.DMA,
        ],
        compiler_params=pltpu.CompilerParams(
            use_tc_tiling_on_sc=True, needs_layout_passes=False),
    )
    def depad(tableT_hbm, tail_hbm, out_hbm, ibuf, obuf, tbuf,
              rsem0, rsem1, rsem2, wsem0, wsem1, wsem2, tsem):
        wid = lax.axis_index("s") * NC + lax.axis_index("c")
        nchunks = BASE_CHUNKS + jnp.where(wid < EXTRA_CHUNKS, 1, 0)
        rsems = (rsem0, rsem1, rsem2)
        wsems = (wsem0, wsem1, wsem2)

        def chunk_id(j):
            return wid + NW * j

        def issue_read(j, slot):
            for b in range(EMBED_DIM // 8):
                pltpu.async_copy(
                    tableT_hbm.at[pl.ds(8 * b, 8),
                                  pl.ds(chunk_id(j) * CHUNK, CHUNK)],
                    ibuf.at[slot, pl.ds(8 * b, 8)], rsems[slot])

        def drain_read(j, slot):
            pltpu.make_async_copy(
                out_hbm.at[pl.ds(0, EMBED_DIM)], ibuf.at[slot],
                rsems[slot]).wait()

        def issue_write(j, slot):
            pltpu.async_copy(
                obuf.at[slot],
                out_hbm.at[pl.ds(chunk_id(j) * (CHUNK // 2), CHUNK // 2)],
                wsems[slot])

        def drain_write(j, slot):
            pltpu.make_async_copy(
                obuf.at[slot],
                out_hbm.at[pl.ds(chunk_id(j) * (CHUNK // 2), CHUNK // 2)],
                wsems[slot]).wait()

        @pl.when(wid == NW - 1)
        def _():
            pltpu.async_copy(tail_hbm, tbuf, tsem).wait()
            pltpu.async_copy(
                tbuf, out_hbm.at[pl.ds(N_FULL_CHUNKS * (CHUNK // 2),
                                       TAIL_ROWS // 2)], tsem).wait()

        row_iota = lax.iota(jnp.int32, 16)
        xor_idx = {k: jnp.bitwise_xor(row_iota, k) for k in (1, 2, 4, 8)}
        lane_masks = {k: (jnp.bitwise_and(row_iota, k) == 0)
                      for k in (1, 2, 4, 8)}

        shuf_dnums = lax.GatherDimensionNumbers(
            offset_dims=(), collapsed_slice_dims=(0,), start_index_map=(0,))

        def shuf(v, k):
            return lax.gather(
                v, xor_idx[k][:, None], shuf_dnums, slice_sizes=(1,),
                mode=lax.GatherScatterMode.PROMISE_IN_BOUNDS)

        for s in range(3):
            @pl.when(nchunks > s)
            def _(s=s):
                issue_read(s, s)

        def process(j, slot):
            drain_read(j, slot)

            @pl.when(j >= 3)
            def _():
                drain_write(j - 3, slot)

            @plsc.parallel_loop(0, CHUNK // 16, unroll=1)
            def tloop(kb):
                for g in range(EMBED_DIM // 16):
                    a = [ibuf[slot, 16 * g + j, pl.ds(16 * kb, 16)]
                         for j in range(16)]
                    for k in (1, 2, 4, 8):
                        m = lane_masks[k]
                        nxt = list(a)
                        for p in range(16):
                            if p & k:
                                continue
                            q = p | k
                            nxt[p] = jnp.where(m, a[p], shuf(a[q], k))
                            nxt[q] = jnp.where(m, shuf(a[p], k), a[q])
                        a = nxt
                    for i in range(16):
                        obuf[slot, 8 * kb + i // 2,
                             pl.ds(EMBED_DIM * (i % 2) + 16 * g, 16)] = a[i]

            issue_write(j, slot)

            @pl.when(j + 3 < nchunks)
            def _():
                issue_read(j + 3, slot)

        @pl.loop(0, BASE_CHUNKS // 3)
        def body(t):
            for s in range(3):
                process(3 * t + s, s)

        process(BASE_CHUNKS // 3 * 3, 0)

        @pl.when(nchunks > BASE_CHUNKS)
        def _():
            process(BASE_CHUNKS // 3 * 3 + 1, 1)

        res = lax.rem(nchunks, 3)
        for r in range(3):
            @pl.when(res == r)
            def _(r=r):
                for t in range(3):
                    drain_write(nchunks - 3 + t, (r + t) % 3)

    return depad(tableT, tail128)


IDX_W = HIST
NBUF = 4


def _sc_pool(x2, table_lin):
    mesh = plsc.VectorSubcoreMesh(core_axis_name="c", subcore_axis_name="s")

    @functools.partial(
        pl.kernel,
        out_type=jax.ShapeDtypeStruct((BATCH, EMBED_DIM), jnp.float32),
        mesh=mesh,
        scratch_types=[
            pltpu.VMEM((ROWS_PER_W, IDX_W), jnp.int32),
            pltpu.VMEM((NBUF, IDX_W, EMBED_DIM), jnp.float32),
            pltpu.VMEM((ROWS_PER_W, EMBED_DIM), jnp.float32),
            pltpu.SemaphoreType.DMA,
            pltpu.SemaphoreType.DMA,
            pltpu.SemaphoreType.DMA,
            pltpu.SemaphoreType.DMA,
        ],
        compiler_params=pltpu.CompilerParams(use_tc_tiling_on_sc=False),
    )
    def pool(x_hbm, table_hbm, out_hbm, idx_v, gbuf, outbuf,
             sem0, sem1, sem2, sem3):
        wid = lax.axis_index("s") * NC + lax.axis_index("c")
        pltpu.sync_copy(
            x_hbm.at[pl.ds(wid * ROWS_PER_W, ROWS_PER_W)], idx_v)
        sems = (sem0, sem1, sem2, sem3)

        def issue(b, slot, sem):
            pltpu.async_copy(
                table_hbm.at[idx_v.at[b]], gbuf.at[slot], sem)

        for s in range(NBUF):
            issue(s, s, sems[s])

        @pl.loop(0, ROWS_PER_W, step=NBUF)
        def body(b):
            for s in range(NBUF):
                bb = b + s
                pltpu.make_async_copy(
                    table_hbm.at[idx_v.at[bb]], gbuf.at[s], sems[s]).wait()

                zero = jnp.zeros((16,), jnp.float32)

                def red(r, carry, s=s):
                    a0, a1, a2, a3 = carry
                    a0 = a0 + gbuf[s, r, 0:16]
                    a1 = a1 + gbuf[s, r, 16:32]
                    a2 = a2 + gbuf[s, r, 32:48]
                    a3 = a3 + gbuf[s, r, 48:64]
                    return (a0, a1, a2, a3)

                a0, a1, a2, a3 = pl.loop(
                    0, IDX_W, init_carry=(zero, zero, zero, zero),
                    unroll=8)(red)
                outbuf[bb, 0:16] = a0
                outbuf[bb, 16:32] = a1
                outbuf[bb, 32:48] = a2
                outbuf[bb, 48:64] = a3

                @pl.when(bb + NBUF < ROWS_PER_W)
                def _(bb=bb, s=s):
                    issue(bb + NBUF, s, sems[s])

        pltpu.sync_copy(outbuf, out_hbm.at[pl.ds(wid * ROWS_PER_W, ROWS_PER_W)])

    return pool(x2, table_lin)


BLK = 1024


def _mlp(x, sums, W1, b1, W2, b2):
    def mlp_body(x_ref, s_ref, w1_ref, b1_ref, w2_ref, b2_ref, o_ref):
        xi = x_ref[...]
        lengths = jnp.sum((xi != 0).astype(jnp.float32), axis=1, keepdims=True)
        pooled = s_ref[...] / lengths
        h = lax.dot_general(
            pooled, w1_ref[...], (((1,), (1,)), ((), ())),
            precision=lax.Precision.HIGHEST,
            preferred_element_type=jnp.float32) + b1_ref[...]
        h = jnp.maximum(h, 0.0)
        o_ref[...] = lax.dot_general(
            h, w2_ref[...], (((1,), (1,)), ((), ())),
            precision=lax.Precision.HIGHEST,
            preferred_element_type=jnp.float32) + b2_ref[...]

    return pl.pallas_call(
        mlp_body,
        grid=(BATCH // BLK,),
        in_specs=[
            pl.BlockSpec((BLK, HIST), lambda i: (i, 0)),
            pl.BlockSpec((BLK, EMBED_DIM), lambda i: (i, 0)),
            pl.BlockSpec((HIDDEN, EMBED_DIM), lambda i: (0, 0)),
            pl.BlockSpec((1, HIDDEN), lambda i: (0, 0)),
            pl.BlockSpec((OUT, HIDDEN), lambda i: (0, 0)),
            pl.BlockSpec((1, OUT), lambda i: (0, 0)),
        ],
        out_specs=pl.BlockSpec((BLK, OUT), lambda i: (i, 0)),
        out_shape=jax.ShapeDtypeStruct((BATCH, OUT), jnp.float32),
    )(x, sums, W1, b1.reshape(1, HIDDEN), W2, b2.reshape(1, OUT))


def kernel(x, table, W1, b1, W2, b2):
    tableT = table.T
    tail128 = table[N_FULL_CHUNKS * CHUNK:].reshape(TAIL_ROWS // 2, 128)
    packed = _sc_depad(tableT, tail128)
    table_lin = packed.reshape(VOCAB, EMBED_DIM)
    sums = _sc_pool(x, table_lin)
    return _mlp(x, sums, W1, b1, W2, b2)
